# baseline (speedup 1.0000x reference)
import jax
import jax.numpy as jnp
from jax import lax
from jax.experimental import pallas as pl
from jax.experimental.pallas import tpu as pltpu
from jax.experimental.pallas import tpu_sc as plsc

BATCH = 16384
EMBED = 32
PACK = 4
NODE = 1000000
NUM_CORES = 2
NUM_SUBCORES = 16
NUM_WORKERS = NUM_CORES * NUM_SUBCORES
B_PER_W = BATCH // NUM_WORKERS
IDX_ROWS = 4
CHUNK = 128
PASS_ROWS = 256
LN2 = 0.6931471805599453


def _log1p_of_exp_neg(az):
    u = jnp.exp(-az)
    y = 1.0 + u
    bits = plsc.bitcast(y, jnp.int32)
    e = (bits >> 23) - 127
    m = plsc.bitcast((bits & 0x007FFFFF) | 0x3F800000, jnp.float32)
    s = (m - 1.0) / (m + 1.0)
    s2 = s * s
    poly = 1.0 + s2 * (1.0 / 3.0 + s2 * (1.0 / 5.0 + s2 * (1.0 / 7.0 + s2 * (1.0 / 9.0))))
    return e.astype(jnp.float32) * LN2 + 2.0 * s * poly


def _sc_body(a_hbm, b_hbm, sign_hbm, emb_hbm, ctx_hbm, out_hbm,
             a_idx, b_idx, a_rec, b_rec, a_r, b_r, out_v, sign_v, sem):
    wid = lax.axis_index("s") * NUM_CORES + lax.axis_index("c")
    base = wid * B_PER_W

    pltpu.sync_copy(a_hbm.at[pl.ds(wid * IDX_ROWS, IDX_ROWS)], a_idx)
    pltpu.sync_copy(b_hbm.at[pl.ds(wid * IDX_ROWS, IDX_ROWS)], b_idx)
    pltpu.sync_copy(sign_hbm, sign_v)

    for j in range(IDX_ROWS):
        for t in range(0, 128, 16):
            a_rec[j, pl.ds(t, 16)] = lax.shift_right_logical(
                a_idx[j, pl.ds(t, 16)], 2)
            b_rec[j, pl.ds(t, 16)] = lax.shift_right_logical(
                b_idx[j, pl.ds(t, 16)], 2)

    lanes = lax.iota(jnp.int32, 16)
    sign_vec = sign_v[...]

    for p in range(B_PER_W // PASS_ROWS):
        copies = []
        for jj in range(PASS_ROWS // CHUNK):
            j = p * (PASS_ROWS // CHUNK) + jj
            copies.append(pltpu.async_copy(
                emb_hbm.at[a_rec.at[j]], a_r.at[pl.ds(jj * CHUNK, CHUNK)], sem))
            copies.append(pltpu.async_copy(
                ctx_hbm.at[b_rec.at[j]], b_r.at[pl.ds(jj * CHUNK, CHUNK)], sem))
        for c in copies:
            c.wait()

        def group_body(g, carry, p=p):
            pos = p * PASS_ROWS + g * 16
            j = lax.shift_right_logical(pos, 7)
            col = pos & 127
            va = a_idx[j, pl.ds(col, 16)]
            vb = b_idx[j, pl.ds(col, 16)]
            off_a = (va & (PACK - 1)) * EMBED
            off_b = (vb & (PACK - 1)) * EMBED
            slot = g * 16 + lanes
            acc = jnp.zeros((16,), jnp.float32)
            for d in range(EMBED):
                av = plsc.load_gather(a_r, [slot, off_a + d])
                bv = plsc.load_gather(b_r, [slot, off_b + d])
                acc = acc + av * bv
            z = -(sign_vec * acc)
            loss = jnp.maximum(z, 0.0) + _log1p_of_exp_neg(jnp.abs(z))
            out_v[pl.ds(pos, 16)] = loss
            return carry

        lax.fori_loop(0, PASS_ROWS // 16, group_body, 0)

    pltpu.sync_copy(out_v, out_hbm.at[pl.ds(base, B_PER_W)])


def kernel(a, b, sign, embeddings, context_embeddings):
    a2 = a.astype(jnp.int32).reshape(NUM_WORKERS * IDX_ROWS, 128)
    b2 = b.astype(jnp.int32).reshape(NUM_WORKERS * IDX_ROWS, 128)
    emb4 = embeddings.reshape(NODE // PACK, PACK * EMBED)
    ctx4 = context_embeddings.reshape(NODE // PACK, PACK * EMBED)
    sign_vec = jnp.broadcast_to(jnp.asarray(sign, jnp.float32), (16,))

    mesh = plsc.VectorSubcoreMesh(core_axis_name="c", subcore_axis_name="s")
    run = pl.kernel(
        _sc_body,
        out_type=jax.ShapeDtypeStruct((BATCH,), jnp.float32),
        mesh=mesh,
        compiler_params=pltpu.CompilerParams(needs_layout_passes=False),
        scratch_types=[
            pltpu.VMEM((IDX_ROWS, 128), jnp.int32),
            pltpu.VMEM((IDX_ROWS, 128), jnp.int32),
            pltpu.VMEM((IDX_ROWS, 128), jnp.int32),
            pltpu.VMEM((IDX_ROWS, 128), jnp.int32),
            pltpu.VMEM((PASS_ROWS, 128), jnp.float32),
            pltpu.VMEM((PASS_ROWS, 128), jnp.float32),
            pltpu.VMEM((B_PER_W,), jnp.float32),
            pltpu.VMEM((16,), jnp.float32),
            pltpu.SemaphoreType.DMA,
        ],
    )
    return run(a2, b2, sign_vec, emb4, ctx4)


# speedup vs baseline: 1.3846x; 1.3846x over previous
import jax
import jax.numpy as jnp
from jax import lax
from jax.experimental import pallas as pl
from jax.experimental.pallas import tpu as pltpu
from jax.experimental.pallas import tpu_sc as plsc

BATCH = 16384
EMBED = 32
GROUP = 8
NUM_CORES = 2
NUM_SUBCORES = 16
NUM_WORKERS = NUM_CORES * NUM_SUBCORES
B_PER_W = BATCH // NUM_WORKERS
IDX_ROWS = 4
CHUNK = 32
NCHUNK = B_PER_W // CHUNK
LN2 = 0.6931471805599453


def _log1p_of_exp_neg(az):
    u = jnp.exp(-az)
    y = 1.0 + u
    bits = plsc.bitcast(y, jnp.int32)
    e = (bits >> 23) - 127
    m = plsc.bitcast((bits & 0x007FFFFF) | 0x3F800000, jnp.float32)
    s = (m - 1.0) / (m + 1.0)
    s2 = s * s
    poly = 1.0 + s2 * (1.0 / 3.0 + s2 * (1.0 / 5.0 + s2 * (1.0 / 7.0 + s2 * (1.0 / 9.0))))
    return e.astype(jnp.float32) * LN2 + 2.0 * s * poly


def _sc_body(a_hbm, b_hbm, sign_hbm, emb_hbm, ctx_hbm, out_hbm,
             a_idx, b_idx, a_t, b_t, out_v, sign_v, sem):
    wid = lax.axis_index("s") * NUM_CORES + lax.axis_index("c")
    base = wid * B_PER_W

    pltpu.sync_copy(a_hbm.at[pl.ds(wid * IDX_ROWS, IDX_ROWS)], a_idx)
    pltpu.sync_copy(b_hbm.at[pl.ds(wid * IDX_ROWS, IDX_ROWS)], b_idx)
    pltpu.sync_copy(sign_hbm, sign_v)

    lanes = lax.iota(jnp.int32, 16)
    sign_vec = sign_v[...]

    def chunk_body(c, carry):
        copies = []
        for g16 in range(CHUNK // 16):
            pos = c * CHUNK + g16 * 16
            j = lax.shift_right_logical(pos, 7)
            col = pos & 127
            va = a_idx[j, pl.ds(col, 16)] & ~(GROUP - 1)
            vb = b_idx[j, pl.ds(col, 16)] & ~(GROUP - 1)
            for r in range(16):
                slot = g16 * 16 + r
                sa = pl.multiple_of(va[r], GROUP)
                sb = pl.multiple_of(vb[r], GROUP)
                copies.append(pltpu.async_copy(
                    emb_hbm.at[pl.ds(sa, GROUP)], a_t.at[slot], sem))
                copies.append(pltpu.async_copy(
                    ctx_hbm.at[pl.ds(sb, GROUP)], b_t.at[slot], sem))
        for cp in copies:
            cp.wait()

        for g in range(CHUNK // 16):
            pos = c * CHUNK + g * 16
            j = lax.shift_right_logical(pos, 7)
            col = pos & 127
            sub_a = a_idx[j, pl.ds(col, 16)] & (GROUP - 1)
            sub_b = b_idx[j, pl.ds(col, 16)] & (GROUP - 1)
            slot = g * 16 + lanes
            acc = jnp.zeros((16,), jnp.float32)
            for d in range(EMBED):
                d_vec = jnp.full((16,), d, jnp.int32)
                av = plsc.load_gather(a_t, [slot, sub_a, d_vec])
                bv = plsc.load_gather(b_t, [slot, sub_b, d_vec])
                acc = acc + av * bv
            z = -(sign_vec * acc)
            loss = jnp.maximum(z, 0.0) + _log1p_of_exp_neg(jnp.abs(z))
            out_v[pl.ds(pos, 16)] = loss
        return carry

    lax.fori_loop(0, NCHUNK, chunk_body, 0)

    pltpu.sync_copy(out_v, out_hbm.at[pl.ds(base, B_PER_W)])


def kernel(a, b, sign, embeddings, context_embeddings):
    a2 = a.astype(jnp.int32).reshape(NUM_WORKERS * IDX_ROWS, 128)
    b2 = b.astype(jnp.int32).reshape(NUM_WORKERS * IDX_ROWS, 128)
    sign_vec = jnp.broadcast_to(jnp.asarray(sign, jnp.float32), (16,))

    mesh = plsc.VectorSubcoreMesh(core_axis_name="c", subcore_axis_name="s")
    run = pl.kernel(
        _sc_body,
        out_type=jax.ShapeDtypeStruct((BATCH,), jnp.float32),
        mesh=mesh,
        compiler_params=pltpu.CompilerParams(needs_layout_passes=False),
        scratch_types=[
            pltpu.VMEM((IDX_ROWS, 128), jnp.int32),
            pltpu.VMEM((IDX_ROWS, 128), jnp.int32),
            pltpu.VMEM((CHUNK, GROUP, EMBED), jnp.float32),
            pltpu.VMEM((CHUNK, GROUP, EMBED), jnp.float32),
            pltpu.VMEM((B_PER_W,), jnp.float32),
            pltpu.VMEM((16,), jnp.float32),
            pltpu.SemaphoreType.DMA,
        ],
    )
    return run(a2, b2, sign_vec, embeddings, context_embeddings)


# speedup vs baseline: 2.2426x; 1.6196x over previous
import jax
import jax.numpy as jnp
from jax import lax
from jax.experimental import pallas as pl
from jax.experimental.pallas import tpu as pltpu
from jax.experimental.pallas import tpu_sc as plsc

BATCH = 16384
EMBED = 32
NODE = 1000000
TILE_ROWS = 8
NUM_CORES = 2
NUM_SUBCORES = 16
NUM_WORKERS = NUM_CORES * NUM_SUBCORES
B_PER_W = BATCH // NUM_WORKERS
IDX_ROWS = 4
CHUNK = 32
NCHUNK = B_PER_W // CHUNK
LN2 = 0.6931471805599453


def _log1p_of_exp_neg(az):
    u = jnp.exp(-az)
    y = 1.0 + u
    bits = plsc.bitcast(y, jnp.int32)
    e = (bits >> 23) - 127
    m = plsc.bitcast((bits & 0x007FFFFF) | 0x3F800000, jnp.float32)
    s = (m - 1.0) / (m + 1.0)
    s2 = s * s
    poly = 1.0 + s2 * (1.0 / 3.0 + s2 * (1.0 / 5.0 + s2 * (1.0 / 7.0 + s2 * (1.0 / 9.0))))
    return e.astype(jnp.float32) * LN2 + 2.0 * s * poly


def _sc_body(a_hbm, b_hbm, sign_hbm, emb_hbm, ctx_hbm, out_hbm,
             a_idx, b_idx, a_rec, b_rec, a_sub, b_sub,
             a_tiles, b_tiles, out_v, sign_v, sem):
    wid = lax.axis_index("s") * NUM_CORES + lax.axis_index("c")
    base = wid * B_PER_W

    pltpu.sync_copy(a_hbm.at[pl.ds(wid * IDX_ROWS, IDX_ROWS)], a_idx)
    pltpu.sync_copy(b_hbm.at[pl.ds(wid * IDX_ROWS, IDX_ROWS)], b_idx)
    pltpu.sync_copy(sign_hbm, sign_v)

    for j in range(IDX_ROWS):
        for t in range(0, 128, 16):
            va = a_idx[j, pl.ds(t, 16)]
            vb = b_idx[j, pl.ds(t, 16)]
            pos = j * 128 + t
            a_rec[pl.ds(pos, 16)] = va >> 3
            b_rec[pl.ds(pos, 16)] = vb >> 3
            a_sub[pl.ds(pos, 16)] = va & 7
            b_sub[pl.ds(pos, 16)] = vb & 7

    lanes = lax.iota(jnp.int32, 16)
    sign_vec = sign_v[...]

    def chunk_body(c, carry):
        copies = []
        for g16 in range(CHUNK // 16):
            va = a_rec[pl.ds(c * CHUNK + g16 * 16, 16)]
            vb = b_rec[pl.ds(c * CHUNK + g16 * 16, 16)]
            for r in range(16):
                slot = g16 * 16 + r
                copies.append(pltpu.async_copy(
                    emb_hbm.at[va[r]], a_tiles.at[slot], sem))
                copies.append(pltpu.async_copy(
                    ctx_hbm.at[vb[r]], b_tiles.at[slot], sem))
        for cp in copies:
            cp.wait()
        for g in range(CHUNK // 16):
            slot = g * 16 + lanes
            pos = c * CHUNK + g * 16
            sub_a = a_sub[pl.ds(pos, 16)]
            sub_b = b_sub[pl.ds(pos, 16)]
            acc = jnp.zeros((16,), jnp.float32)
            for d in range(EMBED):
                d_vec = jnp.full((16,), d, jnp.int32)
                av = plsc.load_gather(a_tiles, [slot, sub_a, d_vec])
                bv = plsc.load_gather(b_tiles, [slot, sub_b, d_vec])
                acc = acc + av * bv
            z = -(sign_vec * acc)
            loss = jnp.maximum(z, 0.0) + _log1p_of_exp_neg(jnp.abs(z))
            out_v[pl.ds(pos, 16)] = loss
        return carry

    lax.fori_loop(0, NCHUNK, chunk_body, 0)

    pltpu.sync_copy(out_v, out_hbm.at[pl.ds(base, B_PER_W)])


def kernel(a, b, sign, embeddings, context_embeddings):
    a2 = a.astype(jnp.int32).reshape(NUM_WORKERS * IDX_ROWS, 128)
    b2 = b.astype(jnp.int32).reshape(NUM_WORKERS * IDX_ROWS, 128)
    emb3 = embeddings.reshape(NODE // TILE_ROWS, TILE_ROWS, EMBED)
    ctx3 = context_embeddings.reshape(NODE // TILE_ROWS, TILE_ROWS, EMBED)
    sign_vec = jnp.broadcast_to(jnp.asarray(sign, jnp.float32), (16,))

    mesh = plsc.VectorSubcoreMesh(core_axis_name="c", subcore_axis_name="s")
    run = pl.kernel(
        _sc_body,
        out_type=jax.ShapeDtypeStruct((BATCH,), jnp.float32),
        mesh=mesh,
        compiler_params=pltpu.CompilerParams(needs_layout_passes=False),
        scratch_types=[
            pltpu.VMEM((IDX_ROWS, 128), jnp.int32),
            pltpu.VMEM((IDX_ROWS, 128), jnp.int32),
            pltpu.VMEM((B_PER_W,), jnp.int32),
            pltpu.VMEM((B_PER_W,), jnp.int32),
            pltpu.VMEM((B_PER_W,), jnp.int32),
            pltpu.VMEM((B_PER_W,), jnp.int32),
            pltpu.VMEM((CHUNK, TILE_ROWS, EMBED), jnp.float32),
            pltpu.VMEM((CHUNK, TILE_ROWS, EMBED), jnp.float32),
            pltpu.VMEM((B_PER_W,), jnp.float32),
            pltpu.VMEM((16,), jnp.float32),
            pltpu.SemaphoreType.DMA,
        ],
    )
    return run(a2, b2, sign_vec, emb3, ctx3)


# speedup vs baseline: 2.3026x; 1.0268x over previous
import jax
import jax.numpy as jnp
from jax import lax
from jax.experimental import pallas as pl
from jax.experimental.pallas import tpu as pltpu
from jax.experimental.pallas import tpu_sc as plsc

BATCH = 16384
EMBED = 32
NODE = 1000000
TILE_ROWS = 8
NUM_CORES = 2
NUM_SUBCORES = 16
NUM_WORKERS = NUM_CORES * NUM_SUBCORES
B_PER_W = BATCH // NUM_WORKERS
IDX_ROWS = 4
CHUNK = 16
NCHUNK = B_PER_W // CHUNK
LN2 = 0.6931471805599453


def _log1p_of_exp_neg(az):
    u = jnp.exp(-az)
    y = 1.0 + u
    bits = plsc.bitcast(y, jnp.int32)
    e = (bits >> 23) - 127
    m = plsc.bitcast((bits & 0x007FFFFF) | 0x3F800000, jnp.float32)
    s = (m - 1.0) / (m + 1.0)
    s2 = s * s
    poly = 1.0 + s2 * (1.0 / 3.0 + s2 * (1.0 / 5.0 + s2 * (1.0 / 7.0 + s2 * (1.0 / 9.0))))
    return e.astype(jnp.float32) * LN2 + 2.0 * s * poly


def _sc_body(a_hbm, b_hbm, sign_hbm, emb_hbm, ctx_hbm, out_hbm,
             a_idx, b_idx, a_rec, b_rec, a_sub, b_sub,
             a_t0, a_t1, b_t0, b_t1, out_v, sign_v, sem0, sem1):
    wid = lax.axis_index("s") * NUM_CORES + lax.axis_index("c")
    base = wid * B_PER_W

    pltpu.sync_copy(a_hbm.at[pl.ds(wid * IDX_ROWS, IDX_ROWS)], a_idx)
    pltpu.sync_copy(b_hbm.at[pl.ds(wid * IDX_ROWS, IDX_ROWS)], b_idx)
    pltpu.sync_copy(sign_hbm, sign_v)

    for j in range(IDX_ROWS):
        for t in range(0, 128, 16):
            va = a_idx[j, pl.ds(t, 16)]
            vb = b_idx[j, pl.ds(t, 16)]
            pos = j * 128 + t
            a_rec[pl.ds(pos, 16)] = va >> 3
            b_rec[pl.ds(pos, 16)] = vb >> 3
            a_sub[pl.ds(pos, 16)] = va & 7
            b_sub[pl.ds(pos, 16)] = vb & 7

    lanes = lax.iota(jnp.int32, 16)
    sign_vec = sign_v[...]

    def fire(c, at, bt, sem):
        va = a_rec[pl.ds(c * CHUNK, 16)]
        vb = b_rec[pl.ds(c * CHUNK, 16)]
        for r in range(16):
            pltpu.async_copy(emb_hbm.at[va[r]], at.at[r], sem)
            pltpu.async_copy(ctx_hbm.at[vb[r]], bt.at[r], sem)

    def drain(at, bt, sem):
        for r in range(16):
            pltpu.make_async_copy(emb_hbm.at[0], at.at[r], sem).wait()
            pltpu.make_async_copy(ctx_hbm.at[0], bt.at[r], sem).wait()

    def compute(c, at, bt):
        pos = c * CHUNK
        sub_a = a_sub[pl.ds(pos, 16)]
        sub_b = b_sub[pl.ds(pos, 16)]
        acc = jnp.zeros((16,), jnp.float32)
        for d in range(EMBED):
            d_vec = jnp.full((16,), d, jnp.int32)
            av = plsc.load_gather(at, [lanes, sub_a, d_vec])
            bv = plsc.load_gather(bt, [lanes, sub_b, d_vec])
            acc = acc + av * bv
        z = -(sign_vec * acc)
        loss = jnp.maximum(z, 0.0) + _log1p_of_exp_neg(jnp.abs(z))
        out_v[pl.ds(pos, 16)] = loss

    fire(0, a_t0, b_t0, sem0)
    fire(1, a_t1, b_t1, sem1)

    def body(i, carry):
        e = i * 2
        drain(a_t0, b_t0, sem0)
        compute(e, a_t0, b_t0)
        fire(e + 2, a_t0, b_t0, sem0)
        drain(a_t1, b_t1, sem1)
        compute(e + 1, a_t1, b_t1)
        fire(e + 3, a_t1, b_t1, sem1)
        return carry

    lax.fori_loop(0, NCHUNK // 2 - 1, body, 0)

    e = NCHUNK - 2
    drain(a_t0, b_t0, sem0)
    compute(e, a_t0, b_t0)
    drain(a_t1, b_t1, sem1)
    compute(e + 1, a_t1, b_t1)

    pltpu.sync_copy(out_v, out_hbm.at[pl.ds(base, B_PER_W)])


def kernel(a, b, sign, embeddings, context_embeddings):
    a2 = a.astype(jnp.int32).reshape(NUM_WORKERS * IDX_ROWS, 128)
    b2 = b.astype(jnp.int32).reshape(NUM_WORKERS * IDX_ROWS, 128)
    emb3 = embeddings.reshape(NODE // TILE_ROWS, TILE_ROWS, EMBED)
    ctx3 = context_embeddings.reshape(NODE // TILE_ROWS, TILE_ROWS, EMBED)
    sign_vec = jnp.broadcast_to(jnp.asarray(sign, jnp.float32), (16,))

    buf = pltpu.VMEM((CHUNK, TILE_ROWS, EMBED), jnp.float32)
    mesh = plsc.VectorSubcoreMesh(core_axis_name="c", subcore_axis_name="s")
    run = pl.kernel(
        _sc_body,
        out_type=jax.ShapeDtypeStruct((BATCH,), jnp.float32),
        mesh=mesh,
        compiler_params=pltpu.CompilerParams(needs_layout_passes=False),
        scratch_types=[
            pltpu.VMEM((IDX_ROWS, 128), jnp.int32),
            pltpu.VMEM((IDX_ROWS, 128), jnp.int32),
            pltpu.VMEM((B_PER_W,), jnp.int32),
            pltpu.VMEM((B_PER_W,), jnp.int32),
            pltpu.VMEM((B_PER_W,), jnp.int32),
            pltpu.VMEM((B_PER_W,), jnp.int32),
            buf, buf, buf, buf,
            pltpu.VMEM((B_PER_W,), jnp.float32),
            pltpu.VMEM((16,), jnp.float32),
            pltpu.SemaphoreType.DMA,
            pltpu.SemaphoreType.DMA,
        ],
    )
    return run(a2, b2, sign_vec, emb3, ctx3)
